# Initial kernel scaffold; baseline (speedup 1.0000x reference)
import jax
import jax.numpy as jnp
from jax.experimental import pallas as pl


def kernel(positions, weight):
    raise NotImplementedError("write your pallas kernel here")


# speedup vs baseline: 1.6079x; 1.6079x over previous
import functools

import jax
import jax.numpy as jnp
from jax import lax
from jax.experimental import pallas as pl
from jax.experimental.pallas import tpu as pltpu
from jax.experimental.pallas import tpu_sc as plsc

NUM_POSITIONS = 8192
EMBEDDING_DIM = 2048
TOTAL = 4 * 8192

NUM_WORKERS = 32
B_PER_W = TOTAL // NUM_WORKERS
K = 16
NBUF = 2
STEPS = B_PER_W // K


def _emb_body(idx_hbm, table_hbm, out_hbm, idx_v, rows_v, gsems):
    nc = plsc.get_sparse_core_info().num_cores
    wid = lax.axis_index("s") * nc + lax.axis_index("c")
    base = wid * B_PER_W

    pltpu.sync_copy(idx_hbm.at[pl.ds(base, B_PER_W)], idx_v)

    def gather_start(step, buf):
        off = pl.multiple_of(step * K, 8)
        return pltpu.make_async_copy(
            table_hbm.at[idx_v.at[pl.ds(off, K)]], rows_v.at[buf], gsems.at[buf]
        )

    gather_start(0, 0).start()

    def body(i, _):
        buf = lax.rem(i, NBUF)
        nxt = lax.rem(i + 1, NBUF)

        @pl.when(i + 1 < STEPS)
        def _():
            gather_start(i + 1, nxt).start()

        gather_start(i, buf).wait()
        off = pl.multiple_of(base + i * K, 8)
        pltpu.sync_copy(rows_v.at[buf], out_hbm.at[pl.ds(off, K)])
        return 0

    lax.fori_loop(0, STEPS, body, 0)


@functools.partial(
    pl.kernel,
    out_type=jax.ShapeDtypeStruct((TOTAL, EMBEDDING_DIM), jnp.float32),
    mesh=plsc.VectorSubcoreMesh(core_axis_name="c", subcore_axis_name="s"),
    scratch_types=[
        pltpu.VMEM((B_PER_W,), jnp.int32),
        pltpu.VMEM((NBUF, K, EMBEDDING_DIM), jnp.float32),
        pltpu.SemaphoreType.DMA((NBUF,)),
    ],
)
def _emb(idx_hbm, table_hbm, out_hbm, idx_v, rows_v, gsems):
    _emb_body(idx_hbm, table_hbm, out_hbm, idx_v, rows_v, gsems)


def kernel(positions, weight):
    flat = positions.reshape(-1)
    out = _emb(flat, weight)
    return out.reshape(positions.shape + (weight.shape[1],))
